# baseline (speedup 1.0000x reference)
import jax
import jax.numpy as jnp
from jax.experimental import pallas as pl

N = 512
L = 64
HID = 128
_NEG = -1e30


def _layernorm(x, g, b):
    mu = jnp.mean(x, axis=-1, keepdims=True)
    var = jnp.mean((x - mu) ** 2, axis=-1, keepdims=True)
    return (x - mu) * jax.lax.rsqrt(var + 1e-5) * g + b


def _mm(a, b):
    return jnp.dot(a, b, preferred_element_type=jnp.float32)


def _mm_rows(xR, w):
    return jax.lax.dot_general(xR, w, (((0,), (0,)), ((), ())),
                               preferred_element_type=jnp.float32)


def _proj_layer(x, p, H, C, x_rows=False):
    proj = _mm_rows if x_rows else _mm
    inv = 1.0 / (C ** 0.5)
    q = (proj(x, p['Wq']) + p['bq']) * inv
    k = proj(x, p['Wk']) + p['bk']
    v = proj(x, p['Wv']) + p['bv']
    skip = proj(x, p['Ws']) + p['bs']
    qWe = q * p['We']
    sub = jax.lax.broadcasted_iota(jnp.int32, (HID, H), 0) // C
    col = jax.lax.broadcasted_iota(jnp.int32, (HID, H), 1)
    tall = _mm(qWe, jnp.where(sub == col, 1.0, 0.0))
    return q, k, v, skip, tall


def _conv_layer(x, maskT, leaT, leaTb, p, H, C, x_rows=False, prj=None):
    q, k, v, skip, tall = prj if prj is not None else _proj_layer(
        x, p, H, C, x_rows)
    ones_col = jnp.ones((N, 1), jnp.float32)
    We = p['We']
    outs = []
    for h in range(H):
        sl = slice(h * C, (h + 1) * C)
        qh = q[:, sl]
        kh = k[:, sl]
        vh = v[:, sl]
        th = tall[:, h:h + 1]
        s = jax.lax.dot_general(qh, kh, (((1,), (1,)), ((), ())),
                                preferred_element_type=jnp.float32)
        s = jnp.where(maskT, s + leaT * th, _NEG)
        amax = jnp.max(s, axis=1, keepdims=True)
        amax = jnp.where(amax > -1e29, amax, 0.0)
        ex = jnp.exp(s - amax).astype(jnp.bfloat16)
        if C < HID:
            vh_aug = jnp.concatenate([vh, ones_col],
                                     axis=1).astype(jnp.bfloat16)
            m = _mm(ex, vh_aug)
            denom = m[:, C:C + 1]
            ohu = m[:, :C]
        else:
            denom = jnp.sum(ex.astype(jnp.float32), axis=1, keepdims=True)
            ohu = _mm(ex, vh.astype(jnp.bfloat16))
        rec = 1.0 / (denom + 1e-16)
        w = jnp.sum((ex * leaTb).astype(jnp.float32), axis=1, keepdims=True)
        outs.append((ohu + w * We[:, sl]) * rec)
    out = outs[0] if H == 1 else jnp.concatenate(outs, axis=1)
    Wb = p['Wb']
    wb_o = Wb[0 * HID:1 * HID, :] + Wb[2 * HID:3 * HID, :]
    wb_s = Wb[1 * HID:2 * HID, :] - Wb[2 * HID:3 * HID, :]
    beta = jax.nn.sigmoid(_mm(out, wb_o) + _mm(skip, wb_s))
    return beta * skip + (1.0 - beta) * out


_CONV_KEYS = ('Wq', 'bq', 'Wk', 'bk', 'Wv', 'bv', 'Ws', 'bs', 'We', 'Wb')


def _body(nC, nD, nid, nod, nloc, nava, lcs, lme, lnp_, adj, ea,
          *refs):
    it = iter(refs)
    convs = []
    for _ in range(3):
        convs.append({kk: next(it)[:] for kk in _CONV_KEYS})
    (gn, bn, Wl0, bl0, Wl1, bl1, gl, bl, Wf0, bf0, Wf1, bf1,
     out_node, out_loc, out_ge, out_lg, out_lat) = list(it)

    nfeatR = jnp.stack([nC[:], nD[:], nid[:], nod[:], nloc[:], nava[:]],
                       axis=0)

    prj0 = _proj_layer(nfeatR, convs[0], 4, HID // 4, x_rows=True)

    maskT = adj[:].T > 0.0
    leaT = jnp.log1p(ea[:].T)
    leaTb = leaT.astype(jnp.bfloat16)

    x = jax.nn.relu(_conv_layer(nfeatR, maskT, leaT, leaTb, convs[0], 4,
                                HID // 4, x_rows=True, prj=prj0))
    x = jax.nn.relu(_conv_layer(x, maskT, leaT, leaTb, convs[1], 4, HID // 4))
    x = _conv_layer(x, maskT, leaT, leaTb, convs[2], 1, HID)
    node_embs = _layernorm(x, gn[:], bn[:])
    out_node[:] = node_embs

    lfeatR = jnp.stack([lcs[:], lme[:], lnp_[:]], axis=0)
    h = jax.nn.relu(_mm_rows(lfeatR, Wl0[:]) + bl0[:])
    h = _mm(h, Wl1[:]) + bl1[:]
    loc_embs = _layernorm(h, gl[:], bl[:])
    out_loc[:] = loc_embs

    graph_emb = jnp.mean(node_embs, axis=0, keepdims=True)
    loc_global = jnp.mean(loc_embs, axis=0, keepdims=True)
    out_ge[:] = graph_emb
    out_lg[:] = loc_global

    Wf0m = Wf0[:]
    z = jax.nn.relu(_mm(graph_emb, Wf0m[:HID]) + _mm(loc_global, Wf0m[HID:])
                    + bf0[:])
    out_lat[:] = _mm(z, Wf1[:]) + bf1[:]


def kernel(nodes_C, nodes_D, nodes_in_degree, nodes_out_degree, nodes_loc,
           nodes_ava, adj, edge_attr, loc_cpu_speed, loc_min_processor_EAT,
           loc_num_processor, params):
    conv_args = []
    for name in ('conv0', 'conv1', 'conv2'):
        p = params[name]
        for kk in _CONV_KEYS:
            conv_args.append(p[kk])

    args = (nodes_C, nodes_D, nodes_in_degree, nodes_out_degree, nodes_loc,
            nodes_ava, loc_cpu_speed, loc_min_processor_EAT,
            loc_num_processor, adj, edge_attr, *conv_args,
            params['ln_node']['g'], params['ln_node']['b'],
            params['loc0']['W'], params['loc0']['b'],
            params['loc1']['W'], params['loc1']['b'],
            params['ln_loc']['g'], params['ln_loc']['b'],
            params['fus0']['W'], params['fus0']['b'],
            params['fus1']['W'], params['fus1']['b'])

    f32 = jnp.float32
    out_shape = [
        jax.ShapeDtypeStruct((N, HID), f32),
        jax.ShapeDtypeStruct((L, HID), f32),
        jax.ShapeDtypeStruct((1, HID), f32),
        jax.ShapeDtypeStruct((1, HID), f32),
        jax.ShapeDtypeStruct((1, HID), f32),
    ]
    node_embs, loc_embs, graph_emb, loc_global, latent = pl.pallas_call(
        _body, out_shape=out_shape)(*args)
    return node_embs, loc_embs, graph_emb, loc_global, latent


# speedup vs baseline: 1.0276x; 1.0276x over previous
import jax
import jax.numpy as jnp
from jax.experimental import pallas as pl

N = 512
L = 64
HID = 128
_NEG = -1e30


def _layernorm(x, g, b):
    mu = jnp.mean(x, axis=-1, keepdims=True)
    var = jnp.mean((x - mu) ** 2, axis=-1, keepdims=True)
    return (x - mu) * jax.lax.rsqrt(var + 1e-5) * g + b


def _mm(a, b):
    return jnp.dot(a, b, preferred_element_type=jnp.float32)


def _mm_rows(xR, w):
    return jax.lax.dot_general(xR, w, (((0,), (0,)), ((), ())),
                               preferred_element_type=jnp.float32)


def _proj_layer(x, p, H, C, x_rows=False):
    proj = _mm_rows if x_rows else _mm
    inv = 1.0 / (C ** 0.5)
    q = (proj(x, p['Wq']) + p['bq']) * inv
    k = proj(x, p['Wk']) + p['bk']
    v = proj(x, p['Wv']) + p['bv']
    skip = proj(x, p['Ws']) + p['bs']
    qWe = q * p['We']
    sub = jax.lax.broadcasted_iota(jnp.int32, (HID, H), 0) // C
    col = jax.lax.broadcasted_iota(jnp.int32, (HID, H), 1)
    tall = _mm(qWe, jnp.where(sub == col, 1.0, 0.0))
    return q, k, v, skip, tall


def _conv_layer(x, maskT, leaT, p, H, C, x_rows=False, prj=None):
    q, k, v, skip, tall = prj if prj is not None else _proj_layer(
        x, p, H, C, x_rows)
    ones_col = jnp.ones((N, 1), jnp.float32)
    We = p['We']
    outs = []
    for h in range(H):
        sl = slice(h * C, (h + 1) * C)
        qh = q[:, sl]
        kh = k[:, sl]
        vh = v[:, sl]
        th = tall[:, h:h + 1]
        s = jax.lax.dot_general(qh, kh, (((1,), (1,)), ((), ())),
                                preferred_element_type=jnp.float32)
        s = jnp.where(maskT, s + leaT * th, _NEG)
        amax = jnp.max(s, axis=1, keepdims=True)
        amax = jnp.where(amax > -1e29, amax, 0.0)
        ex = jnp.exp(s - amax)
        if C < HID:
            vh_aug = jnp.concatenate([vh, ones_col], axis=1)
            m = _mm(ex, vh_aug)
            denom = m[:, C:C + 1]
            ohu = m[:, :C]
        else:
            denom = jnp.sum(ex, axis=1, keepdims=True)
            ohu = _mm(ex, vh)
        rec = 1.0 / (denom + 1e-16)
        w = jnp.sum(ex * leaT, axis=1, keepdims=True)
        outs.append((ohu + w * We[:, sl]) * rec)
    out = outs[0] if H == 1 else jnp.concatenate(outs, axis=1)
    Wb = p['Wb']
    wb_o = Wb[0 * HID:1 * HID, :] + Wb[2 * HID:3 * HID, :]
    wb_s = Wb[1 * HID:2 * HID, :] - Wb[2 * HID:3 * HID, :]
    beta = jax.nn.sigmoid(_mm(out, wb_o) + _mm(skip, wb_s))
    return beta * skip + (1.0 - beta) * out


_CONV_KEYS = ('Wq', 'bq', 'Wk', 'bk', 'Wv', 'bv', 'Ws', 'bs', 'We', 'Wb')


def _body(nC, nD, nid, nod, nloc, nava, lcs, lme, lnp_, adj, ea,
          *refs):
    it = iter(refs)
    convs = []
    for _ in range(3):
        convs.append({kk: next(it)[:] for kk in _CONV_KEYS})
    (gn, bn, Wl0, bl0, Wl1, bl1, gl, bl, Wf0, bf0, Wf1, bf1,
     out_node, out_loc, out_ge, out_lg, out_lat) = list(it)

    nfeatR = jnp.stack([nC[:], nD[:], nid[:], nod[:], nloc[:], nava[:]],
                       axis=0)

    prj0 = _proj_layer(nfeatR, convs[0], 4, HID // 4, x_rows=True)

    maskT = adj[:].T > 0.0
    leaT = jnp.log1p(ea[:].T)

    x = jax.nn.relu(_conv_layer(nfeatR, maskT, leaT, convs[0], 4, HID // 4,
                                x_rows=True, prj=prj0))
    x = jax.nn.relu(_conv_layer(x, maskT, leaT, convs[1], 4, HID // 4))
    x = _conv_layer(x, maskT, leaT, convs[2], 1, HID)
    node_embs = _layernorm(x, gn[:], bn[:])
    out_node[:] = node_embs

    lfeatR = jnp.stack([lcs[:], lme[:], lnp_[:]], axis=0)
    h = jax.nn.relu(_mm_rows(lfeatR, Wl0[:]) + bl0[:])
    h = _mm(h, Wl1[:]) + bl1[:]
    loc_embs = _layernorm(h, gl[:], bl[:])
    out_loc[:] = loc_embs

    graph_emb = jnp.mean(node_embs, axis=0, keepdims=True)
    loc_global = jnp.mean(loc_embs, axis=0, keepdims=True)
    out_ge[:] = graph_emb
    out_lg[:] = loc_global

    Wf0m = Wf0[:]
    z = jax.nn.relu(_mm(graph_emb, Wf0m[:HID]) + _mm(loc_global, Wf0m[HID:])
                    + bf0[:])
    out_lat[:] = _mm(z, Wf1[:]) + bf1[:]


def kernel(nodes_C, nodes_D, nodes_in_degree, nodes_out_degree, nodes_loc,
           nodes_ava, adj, edge_attr, loc_cpu_speed, loc_min_processor_EAT,
           loc_num_processor, params):
    conv_args = []
    for name in ('conv0', 'conv1', 'conv2'):
        p = params[name]
        for kk in _CONV_KEYS:
            conv_args.append(p[kk])

    args = (nodes_C, nodes_D, nodes_in_degree, nodes_out_degree, nodes_loc,
            nodes_ava, loc_cpu_speed, loc_min_processor_EAT,
            loc_num_processor, adj, edge_attr, *conv_args,
            params['ln_node']['g'], params['ln_node']['b'],
            params['loc0']['W'], params['loc0']['b'],
            params['loc1']['W'], params['loc1']['b'],
            params['ln_loc']['g'], params['ln_loc']['b'],
            params['fus0']['W'], params['fus0']['b'],
            params['fus1']['W'], params['fus1']['b'])

    f32 = jnp.float32
    out_shape = [
        jax.ShapeDtypeStruct((N, HID), f32),
        jax.ShapeDtypeStruct((L, HID), f32),
        jax.ShapeDtypeStruct((1, HID), f32),
        jax.ShapeDtypeStruct((1, HID), f32),
        jax.ShapeDtypeStruct((1, HID), f32),
    ]
    node_embs, loc_embs, graph_emb, loc_global, latent = pl.pallas_call(
        _body, out_shape=out_shape)(*args)
    return node_embs, loc_embs, graph_emb, loc_global, latent
